# baseline (speedup 1.0000x reference)
import functools

import jax
import jax.numpy as jnp
from jax import lax
from jax.experimental import pallas as pl
from jax.experimental.pallas import tpu as pltpu
from jax.experimental.pallas import tpu_sc as plsc

VOCAB = 1000
BATCH = 4096
SEQ = 20
SEQP = 24
D = VOCAB
DP = 1024
B = BATCH * SEQP
NPART = 4
PB = BATCH // NPART
BP = PB * SEQP
NC = 2
NS = 16
NW = NC * NS
BPW = BP // NW
CHUNK = 24
NCHUNK = BPW // CHUNK
NBUF = 4


def _make_gather():
    mesh = plsc.VectorSubcoreMesh(core_axis_name="c", subcore_axis_name="s")

    @functools.partial(
        pl.kernel,
        mesh=mesh,
        out_type=jax.ShapeDtypeStruct((BP, DP), jnp.float32),
        scratch_types=[
            pltpu.VMEM((BPW,), jnp.int32),
            pltpu.VMEM((CHUNK, DP), jnp.float32),
            pltpu.VMEM((CHUNK, DP), jnp.float32),
            pltpu.VMEM((CHUNK, DP), jnp.float32),
            pltpu.VMEM((CHUNK, DP), jnp.float32),
            pltpu.SemaphoreType.DMA,
            pltpu.SemaphoreType.DMA,
            pltpu.SemaphoreType.DMA,
            pltpu.SemaphoreType.DMA,
            pltpu.SemaphoreType.DMA,
            pltpu.SemaphoreType.DMA,
            pltpu.SemaphoreType.DMA,
            pltpu.SemaphoreType.DMA,
        ],
    )
    def gather_kernel(
        idx_hbm, table_hbm, out_hbm,
        idx_v, rows0, rows1, rows2, rows3, g0, g1, g2, g3, w0, w1, w2, w3,
    ):
        wid = lax.axis_index("s") * NC + lax.axis_index("c")
        base = wid * BPW
        bufs = (rows0, rows1, rows2, rows3)
        gsems = (g0, g1, g2, g3)
        wsems = (w0, w1, w2, w3)
        pltpu.sync_copy(idx_hbm.at[pl.ds(base, BPW)], idx_v)

        def start_gather(c, b):
            pltpu.async_copy(
                table_hbm.at[idx_v.at[pl.ds(c * CHUNK, CHUNK)]], bufs[b], gsems[b]
            )

        def wait_gather(b):
            pltpu.make_async_copy(
                table_hbm.at[idx_v.at[pl.ds(0, CHUNK)]], bufs[b], gsems[b]
            ).wait()

        def start_write(c, b):
            pltpu.async_copy(bufs[b], out_hbm.at[pl.ds(base + c * CHUNK, CHUNK)], wsems[b])

        def wait_write(b):
            pltpu.make_async_copy(bufs[b], out_hbm.at[pl.ds(base, CHUNK)], wsems[b]).wait()

        start_gather(0, 0)
        start_gather(1, 1)
        start_gather(2, 2)
        start_gather(3, 3)
        wait_gather(0)
        start_write(0, 0)

        def body(c):
            for u in range(NBUF):
                r = (1 + u) % NBUF
                rn = u % NBUF
                wait_write(rn)
                start_gather(c + u + 3, rn)
                wait_gather(r)
                start_write(c + u, r)

        pl.loop(1, NCHUNK - 3, step=NBUF)(body)

        for cc in (NCHUNK - 3, NCHUNK - 2, NCHUNK - 1):
            r = cc % NBUF
            wait_gather(r)
            start_write(cc, r)

        for b in range(NBUF):
            wait_write(b)

    return gather_kernel


_gather = _make_gather()


@jax.jit
def kernel(idx, table):
    idx_p = jnp.pad(idx.astype(jnp.int32), ((0, 0), (0, SEQP - SEQ)), mode="edge")
    table_p = jnp.pad(table, ((0, 0), (0, DP - D)))
    idx_flat = idx_p.reshape(B)
    one = jax.lax.optimization_barrier(jnp.float32(1.0))
    acc = jnp.zeros((BATCH, SEQ, D), jnp.float32)
    for k in range(NPART):
        part = _gather(idx_flat[k * BP:(k + 1) * BP], table_p)
        part = part.reshape(PB, SEQP, DP)[:, :SEQ, :D] * one
        acc = lax.dynamic_update_slice(acc, part, (k * PB, 0, 0))
    return acc


# speedup vs baseline: 1.6092x; 1.6092x over previous
import functools

import jax
import jax.numpy as jnp
from jax import lax
from jax.experimental import pallas as pl
from jax.experimental.pallas import tpu as pltpu
from jax.experimental.pallas import tpu_sc as plsc

VOCAB = 1000
BATCH = 4096
SEQ = 20
SEQP = 24
D = VOCAB
DP = 1024
B = BATCH * SEQP
NC = 2
NS = 16
NW = NC * NS
BPW = B // NW
CHUNK = 32
NCHUNK = BPW // CHUNK


def _make_gather():
    mesh = plsc.VectorSubcoreMesh(core_axis_name="c", subcore_axis_name="s")

    @functools.partial(
        pl.kernel,
        mesh=mesh,
        out_type=jax.ShapeDtypeStruct((B, DP), jnp.float32),
        scratch_types=[
            pltpu.VMEM((BPW,), jnp.int32),
            pltpu.VMEM((CHUNK, DP), jnp.float32),
            pltpu.VMEM((CHUNK, DP), jnp.float32),
            pltpu.VMEM((CHUNK, DP), jnp.float32),
            pltpu.SemaphoreType.DMA,
            pltpu.SemaphoreType.DMA,
            pltpu.SemaphoreType.DMA,
            pltpu.SemaphoreType.DMA,
            pltpu.SemaphoreType.DMA,
            pltpu.SemaphoreType.DMA,
        ],
    )
    def gather_kernel(
        idx_hbm, table_hbm, out_hbm,
        idx_v, rows0, rows1, rows2, g0, g1, g2, w0, w1, w2,
    ):
        wid = lax.axis_index("s") * NC + lax.axis_index("c")
        base = wid * BPW
        bufs = (rows0, rows1, rows2)
        gsems = (g0, g1, g2)
        wsems = (w0, w1, w2)
        pltpu.sync_copy(idx_hbm.at[pl.ds(base, BPW)], idx_v)

        def start_gather(c, b):
            pltpu.async_copy(
                table_hbm.at[idx_v.at[pl.ds(c * CHUNK, CHUNK)]], bufs[b], gsems[b]
            )

        def wait_gather(b):
            pltpu.make_async_copy(
                table_hbm.at[idx_v.at[pl.ds(0, CHUNK)]], bufs[b], gsems[b]
            ).wait()

        def start_write(c, b):
            pltpu.async_copy(bufs[b], out_hbm.at[pl.ds(base + c * CHUNK, CHUNK)], wsems[b])

        def wait_write(b):
            pltpu.make_async_copy(bufs[b], out_hbm.at[pl.ds(base, CHUNK)], wsems[b]).wait()

        start_gather(0, 0)
        start_gather(1, 1)
        start_gather(2, 2)
        wait_gather(0)
        start_write(0, 0)

        def body(c):
            for u in range(3):
                r = (1 + u) % 3
                rn = u % 3
                wait_write(rn)
                start_gather(c + u + 2, rn)
                wait_gather(r)
                start_write(c + u, r)

        pl.loop(1, NCHUNK - 2, step=3)(body)

        for cc in (NCHUNK - 2, NCHUNK - 1):
            r = cc % 3
            wait_gather(r)
            start_write(cc, r)

        for b in range(3):
            wait_write(b)

    return gather_kernel


_gather = _make_gather()


@jax.jit
def kernel(idx, table):
    idx_p = jnp.pad(idx.astype(jnp.int32), ((0, 0), (0, SEQP - SEQ)), mode="edge")
    table_p = jnp.pad(table, ((0, 0), (0, DP - D)))
    out = _gather(idx_p.reshape(B), table_p)
    return out.reshape(BATCH, SEQP, DP)[:, :SEQ, :D]
